# baseline (speedup 1.0000x reference)
import jax
import jax.numpy as jnp
from jax.experimental import pallas as pl
from jax.experimental.pallas import tpu as pltpu

B, N, F_IN, S, CH = 4, 64, 32, 4, 32
E = B * N * N

_SECTIONS = [
    ("c1_w0", (S, 64)), ("c1_b0", (1, 64)), ("c1_w1", (64, 32)),
    ("c1_b1", (1, 32)), ("c1_w2jfc", (32 * F_IN, CH)), ("c1_b2rT", (F_IN, CH)),
    ("c1_root", (F_IN, CH)), ("c1_bias", (1, CH)),
    ("c2_w0", (S, 64)), ("c2_b0", (1, 64)), ("c2_w1", (64, 32)),
    ("c2_b1", (1, 32)), ("c2_w2jfc", (32 * CH, CH)), ("c2_b2rT", (CH, CH)),
    ("c2_root", (CH, CH)), ("c2_bias", (1, CH)),
    ("d_w", (CH, 64)), ("d_b", (1, 64)), ("o_w", (64, 10)), ("o_b", (1, 10)),
]
_SHAPES = dict(_SECTIONS)
_OFFSETS = {}
_R = 0
for _name, _shp in _SECTIONS:
    _OFFSETS[_name] = _R
    _R += _shp[0]


def _net_kernel(eT_ref, a_ref, x_ref, w_ref, out_ref):
    eT = eT_ref[:]
    a = a_ref[:]
    xf = x_ref[:, :, :F_IN].reshape(B * N, F_IN)
    mask = x_ref[:, :, F_IN:F_IN + 1]

    def w(name):
        r0 = _OFFSETS[name]
        nr, nc = _SHAPES[name]
        return w_ref[r0:r0 + nr, :nc]

    aT = jnp.transpose(a, (0, 2, 1)).reshape(B * N, N)

    def ecc(feats, p):
        Fc = feats.shape[-1]
        h1 = jax.nn.relu(
            jax.lax.dot(eT, w(p + "w0"), preferred_element_type=jnp.float32)
            + w(p + "b0"))
        h2 = jax.nn.relu(
            jax.lax.dot(h1, w(p + "w1"), preferred_element_type=jnp.float32)
            + w(p + "b1"))
        H = h2.reshape(B * N, N, 32)
        w2jfc = w(p + "w2jfc").reshape(32, Fc, CH)
        G = jax.lax.dot_general(
            feats, w2jfc, (((1,), (1,)), ((), ())),
            preferred_element_type=jnp.float32)
        U = jax.lax.dot_general(
            G, H, (((1,), (2,)), ((0,), (0,))),
            preferred_element_type=jnp.float32)
        Um = U * aT[:, None, :]
        msg = Um.reshape(B, N, CH, N).sum(axis=1)
        msg = jnp.transpose(msg, (0, 2, 1))
        bterm = jax.lax.dot(
            feats, w(p + "b2rT"), preferred_element_type=jnp.float32)
        bmat = jax.lax.dot_general(
            a, bterm.reshape(B, N, CH), (((2,), (1,)), ((0,), (0,))),
            preferred_element_type=jnp.float32)
        rootterm = jax.lax.dot(
            feats, w(p + "root"), preferred_element_type=jnp.float32)
        return msg + bmat + rootterm.reshape(B, N, CH) + w(p + "bias")

    h = ecc(xf, "c1_")
    h = jnp.where(h > 0, h, 0.05 * h)
    h = ecc(h.reshape(B * N, CH), "c2_")
    h = jax.nn.relu(h)

    denom = jnp.clip(jnp.sum(mask, axis=1), 1.0, None)
    pooled = jnp.sum(h * mask, axis=1) / denom
    dh = jax.nn.relu(
        jax.lax.dot(pooled, w("d_w"), preferred_element_type=jnp.float32)
        + w("d_b"))
    logits = (jax.lax.dot(dh, w("o_w"), preferred_element_type=jnp.float32)
              + w("o_b"))
    m = jnp.max(logits, axis=-1, keepdims=True)
    ex = jnp.exp(logits - m)
    out_ref[:] = ex / jnp.sum(ex, axis=-1, keepdims=True)


def kernel(x, a, e, c1_w0, c1_b0, c1_w1, c1_b1, c1_w2, c1_b2, c1_root,
           c1_bias, c2_w0, c2_b0, c2_w1, c2_b1, c2_w2, c2_b2, c2_root,
           c2_bias, d_w, d_b, o_w, o_b):
    eT = e.transpose(0, 2, 1, 3).reshape(E, S)

    vals = {
        "c1_w0": c1_w0, "c1_b0": c1_b0.reshape(1, 64),
        "c1_w1": c1_w1, "c1_b1": c1_b1.reshape(1, 32),
        "c1_w2jfc": c1_w2.reshape(32, CH, F_IN).transpose(0, 2, 1)
                         .reshape(32 * F_IN, CH),
        "c1_b2rT": c1_b2.reshape(CH, F_IN).T,
        "c1_root": c1_root, "c1_bias": c1_bias.reshape(1, CH),
        "c2_w0": c2_w0, "c2_b0": c2_b0.reshape(1, 64),
        "c2_w1": c2_w1, "c2_b1": c2_b1.reshape(1, 32),
        "c2_w2jfc": c2_w2.reshape(32, CH, CH).transpose(0, 2, 1)
                         .reshape(32 * CH, CH),
        "c2_b2rT": c2_b2.reshape(CH, CH).T,
        "c2_root": c2_root, "c2_bias": c2_bias.reshape(1, CH),
        "d_w": d_w, "d_b": d_b.reshape(1, 64),
        "o_w": o_w, "o_b": o_b.reshape(1, 10),
    }
    rows = []
    for name, shp in _SECTIONS:
        v = vals[name]
        if shp[1] < 64:
            v = jnp.pad(v, ((0, 0), (0, 64 - shp[1])))
        rows.append(v)
    packed = jnp.concatenate(rows, axis=0)

    return pl.pallas_call(
        _net_kernel,
        out_shape=jax.ShapeDtypeStruct((B, 10), jnp.float32),
        in_specs=[pl.BlockSpec(memory_space=pltpu.VMEM)] * 4,
        out_specs=pl.BlockSpec(memory_space=pltpu.VMEM),
    )(eT, a, x, packed)


# speedup vs baseline: 1.0429x; 1.0429x over previous
import jax
import jax.numpy as jnp
from jax.experimental import pallas as pl
from jax.experimental.pallas import tpu as pltpu

B, N, F_IN, S, CH = 4, 64, 32, 4, 32
E = B * N * N

_SECTIONS = [
    ("c1_w0", (S, 64)), ("c1_b0", (1, 64)), ("c1_w1", (64, 32)),
    ("c1_b1", (1, 32)), ("c1_w2jfc", (32 * F_IN, CH)), ("c1_b2rT", (F_IN, CH)),
    ("c1_root", (F_IN, CH)), ("c1_bias", (1, CH)),
    ("c2_w0", (S, 64)), ("c2_b0", (1, 64)), ("c2_w1", (64, 32)),
    ("c2_b1", (1, 32)), ("c2_w2jfc", (32 * CH, CH)), ("c2_b2rT", (CH, CH)),
    ("c2_root", (CH, CH)), ("c2_bias", (1, CH)),
    ("d_w", (CH, 64)), ("d_b", (1, 64)), ("o_w", (64, 10)), ("o_b", (1, 10)),
]
_SHAPES = dict(_SECTIONS)
_OFFSETS = {}
_R = 0
for _name, _shp in _SECTIONS:
    _OFFSETS[_name] = _R
    _R += _shp[0]


def _net_kernel(eT_ref, a_ref, x_ref, w_ref, out_ref):
    eT = eT_ref[:]
    a = a_ref[:]
    xf = x_ref[:, :, :F_IN].reshape(B * N, F_IN)
    mask = x_ref[:, :, F_IN:F_IN + 1]

    def w(name):
        r0 = _OFFSETS[name]
        nr, nc = _SHAPES[name]
        return w_ref[r0:r0 + nr, :nc]

    aT = jnp.transpose(a, (0, 2, 1)).reshape(B * N, N)

    def ecc(feats, p):
        Fc = feats.shape[-1]
        bf = jnp.bfloat16
        h1 = jax.nn.relu(
            jax.lax.dot(eT.astype(bf), w(p + "w0").astype(bf),
                        preferred_element_type=jnp.float32)
            + w(p + "b0"))
        h2 = jax.nn.relu(
            jax.lax.dot(h1.astype(bf), w(p + "w1").astype(bf),
                        preferred_element_type=jnp.float32)
            + w(p + "b1"))
        H = jnp.swapaxes(h2.astype(bf).reshape(B, N, N, 32), 1, 2)
        H = H.reshape(B * N, N, 32)
        w2jfc = w(p + "w2jfc").reshape(32, Fc, CH)
        G = jax.lax.dot_general(
            feats.astype(bf), w2jfc.astype(bf), (((1,), (1,)), ((), ())),
            preferred_element_type=jnp.float32)
        U = jax.lax.dot_general(
            G.astype(bf), H, (((1,), (2,)), ((0,), (0,))),
            preferred_element_type=jnp.float32)
        Um = U * aT[:, None, :]
        msg = Um.reshape(B, N, CH, N).sum(axis=1)
        msg = jnp.transpose(msg, (0, 2, 1))
        bterm = jax.lax.dot(
            feats, w(p + "b2rT"), preferred_element_type=jnp.float32)
        bmat = jax.lax.dot_general(
            a, bterm.reshape(B, N, CH), (((2,), (1,)), ((0,), (0,))),
            preferred_element_type=jnp.float32)
        rootterm = jax.lax.dot(
            feats, w(p + "root"), preferred_element_type=jnp.float32)
        return msg + bmat + rootterm.reshape(B, N, CH) + w(p + "bias")

    h = ecc(xf, "c1_")
    h = jnp.where(h > 0, h, 0.05 * h)
    h = ecc(h.reshape(B * N, CH), "c2_")
    h = jax.nn.relu(h)

    denom = jnp.clip(jnp.sum(mask, axis=1), 1.0, None)
    pooled = jnp.sum(h * mask, axis=1) / denom
    dh = jax.nn.relu(
        jax.lax.dot(pooled, w("d_w"), preferred_element_type=jnp.float32)
        + w("d_b"))
    logits = (jax.lax.dot(dh, w("o_w"), preferred_element_type=jnp.float32)
              + w("o_b"))
    m = jnp.max(logits, axis=-1, keepdims=True)
    ex = jnp.exp(logits - m)
    out_ref[:] = ex / jnp.sum(ex, axis=-1, keepdims=True)


def kernel(x, a, e, c1_w0, c1_b0, c1_w1, c1_b1, c1_w2, c1_b2, c1_root,
           c1_bias, c2_w0, c2_b0, c2_w1, c2_b1, c2_w2, c2_b2, c2_root,
           c2_bias, d_w, d_b, o_w, o_b):
    eT = e.reshape(E, S)

    vals = {
        "c1_w0": c1_w0, "c1_b0": c1_b0.reshape(1, 64),
        "c1_w1": c1_w1, "c1_b1": c1_b1.reshape(1, 32),
        "c1_w2jfc": c1_w2.reshape(32, CH, F_IN).transpose(0, 2, 1)
                         .reshape(32 * F_IN, CH),
        "c1_b2rT": c1_b2.reshape(CH, F_IN).T,
        "c1_root": c1_root, "c1_bias": c1_bias.reshape(1, CH),
        "c2_w0": c2_w0, "c2_b0": c2_b0.reshape(1, 64),
        "c2_w1": c2_w1, "c2_b1": c2_b1.reshape(1, 32),
        "c2_w2jfc": c2_w2.reshape(32, CH, CH).transpose(0, 2, 1)
                         .reshape(32 * CH, CH),
        "c2_b2rT": c2_b2.reshape(CH, CH).T,
        "c2_root": c2_root, "c2_bias": c2_bias.reshape(1, CH),
        "d_w": d_w, "d_b": d_b.reshape(1, 64),
        "o_w": o_w, "o_b": o_b.reshape(1, 10),
    }
    rows = []
    for name, shp in _SECTIONS:
        v = vals[name]
        if shp[1] < 64:
            v = jnp.pad(v, ((0, 0), (0, 64 - shp[1])))
        rows.append(v)
    packed = jnp.concatenate(rows, axis=0)

    return pl.pallas_call(
        _net_kernel,
        out_shape=jax.ShapeDtypeStruct((B, 10), jnp.float32),
        in_specs=[pl.BlockSpec(memory_space=pltpu.VMEM)] * 4,
        out_specs=pl.BlockSpec(memory_space=pltpu.VMEM),
    )(eT, a, x, packed)


# speedup vs baseline: 1.0503x; 1.0071x over previous
import jax
import jax.numpy as jnp
from jax.experimental import pallas as pl
from jax.experimental.pallas import tpu as pltpu

B, N, F_IN, S, CH = 4, 64, 32, 4, 32
E = B * N * N

_SECTIONS = [
    ("c1_w0", (S, 64)), ("c1_b0", (1, 64)), ("c1_w1", (64, 32)),
    ("c1_b1", (1, 32)), ("c1_w2jfc", (32 * F_IN, CH)), ("c1_b2rT", (F_IN, CH)),
    ("c1_root", (F_IN, CH)), ("c1_bias", (1, CH)),
    ("c2_w0", (S, 64)), ("c2_b0", (1, 64)), ("c2_w1", (64, 32)),
    ("c2_b1", (1, 32)), ("c2_w2jfc", (32 * CH, CH)), ("c2_b2rT", (CH, CH)),
    ("c2_root", (CH, CH)), ("c2_bias", (1, CH)),
    ("d_w", (CH, 64)), ("d_b", (1, 64)), ("o_w", (64, 10)), ("o_b", (1, 10)),
]
_SHAPES = dict(_SECTIONS)
_OFFSETS = {}
_R = 0
for _name, _shp in _SECTIONS:
    _OFFSETS[_name] = _R
    _R += _shp[0]


def _net_kernel(eW_ref, a_ref, x_ref, w_ref, out_ref):
    eW = eW_ref[:]
    a = a_ref[:]
    xf = x_ref[:, :, :F_IN].reshape(B * N, F_IN)
    mask = x_ref[:, :, F_IN:F_IN + 1]

    def w(name):
        r0 = _OFFSETS[name]
        nr, nc = _SHAPES[name]
        return w_ref[r0:r0 + nr, :nc]

    aT = jnp.transpose(a, (0, 2, 1)).reshape(B * N, N)

    ri = jax.lax.broadcasted_iota(jnp.int32, (4 * 32, 32, 64), 0) // S
    ci = jax.lax.broadcasted_iota(jnp.int32, (4 * 32, 32, 64), 1)

    def edge_mlp_in(w0):
        w0big = jnp.broadcast_to(jnp.tile(w0, (32, 1))[:, None, :],
                                 (128, 32, 64))
        return jnp.where(ri == ci, w0big, 0.0)

    def ecc(feats, p):
        Fc = feats.shape[-1]
        bf = jnp.bfloat16
        h1 = jax.nn.relu(
            jax.lax.dot_general(
                eW.astype(bf), edge_mlp_in(w(p + "w0")).astype(bf),
                (((1,), (0,)), ((), ())),
                preferred_element_type=jnp.float32).reshape(E, 64)
            + w(p + "b0"))
        h2 = jax.nn.relu(
            jax.lax.dot(h1.astype(bf), w(p + "w1").astype(bf),
                        preferred_element_type=jnp.float32)
            + w(p + "b1"))
        H = jnp.swapaxes(h2.astype(bf).reshape(B, N, N, 32), 1, 2)
        H = H.reshape(B * N, N, 32)
        w2jfc = w(p + "w2jfc").reshape(32, Fc, CH)
        G = jax.lax.dot_general(
            feats.astype(bf), w2jfc.astype(bf), (((1,), (1,)), ((), ())),
            preferred_element_type=jnp.float32)
        U = jax.lax.dot_general(
            G.astype(bf), H, (((1,), (2,)), ((0,), (0,))),
            preferred_element_type=jnp.float32)
        Um = U * aT[:, None, :]
        msg = Um.reshape(B, N, CH, N).sum(axis=1)
        msg = jnp.transpose(msg, (0, 2, 1))
        bterm = jax.lax.dot(
            feats, w(p + "b2rT"), preferred_element_type=jnp.float32)
        bmat = jax.lax.dot_general(
            a, bterm.reshape(B, N, CH), (((2,), (1,)), ((0,), (0,))),
            preferred_element_type=jnp.float32)
        rootterm = jax.lax.dot(
            feats, w(p + "root"), preferred_element_type=jnp.float32)
        return msg + bmat + rootterm.reshape(B, N, CH) + w(p + "bias")

    h = ecc(xf, "c1_")
    h = jnp.where(h > 0, h, 0.05 * h)
    h = ecc(h.reshape(B * N, CH), "c2_")
    h = jax.nn.relu(h)

    denom = jnp.clip(jnp.sum(mask, axis=1), 1.0, None)
    pooled = jnp.sum(h * mask, axis=1) / denom
    dh = jax.nn.relu(
        jax.lax.dot(pooled, w("d_w"), preferred_element_type=jnp.float32)
        + w("d_b"))
    logits = (jax.lax.dot(dh, w("o_w"), preferred_element_type=jnp.float32)
              + w("o_b"))
    m = jnp.max(logits, axis=-1, keepdims=True)
    ex = jnp.exp(logits - m)
    out_ref[:] = ex / jnp.sum(ex, axis=-1, keepdims=True)


def kernel(x, a, e, c1_w0, c1_b0, c1_w1, c1_b1, c1_w2, c1_b2, c1_root,
           c1_bias, c2_w0, c2_b0, c2_w1, c2_b1, c2_w2, c2_b2, c2_root,
           c2_bias, d_w, d_b, o_w, o_b):
    eW = e.reshape(E // 32, 32 * S)

    vals = {
        "c1_w0": c1_w0, "c1_b0": c1_b0.reshape(1, 64),
        "c1_w1": c1_w1, "c1_b1": c1_b1.reshape(1, 32),
        "c1_w2jfc": c1_w2.reshape(32, CH, F_IN).transpose(0, 2, 1)
                         .reshape(32 * F_IN, CH),
        "c1_b2rT": c1_b2.reshape(CH, F_IN).T,
        "c1_root": c1_root, "c1_bias": c1_bias.reshape(1, CH),
        "c2_w0": c2_w0, "c2_b0": c2_b0.reshape(1, 64),
        "c2_w1": c2_w1, "c2_b1": c2_b1.reshape(1, 32),
        "c2_w2jfc": c2_w2.reshape(32, CH, CH).transpose(0, 2, 1)
                         .reshape(32 * CH, CH),
        "c2_b2rT": c2_b2.reshape(CH, CH).T,
        "c2_root": c2_root, "c2_bias": c2_bias.reshape(1, CH),
        "d_w": d_w, "d_b": d_b.reshape(1, 64),
        "o_w": o_w, "o_b": o_b.reshape(1, 10),
    }
    rows = []
    for name, shp in _SECTIONS:
        v = vals[name]
        if shp[1] < 64:
            v = jnp.pad(v, ((0, 0), (0, 64 - shp[1])))
        rows.append(v)
    packed = jnp.concatenate(rows, axis=0)

    return pl.pallas_call(
        _net_kernel,
        out_shape=jax.ShapeDtypeStruct((B, 10), jnp.float32),
        in_specs=[pl.BlockSpec(memory_space=pltpu.VMEM)] * 4,
        out_specs=pl.BlockSpec(memory_space=pltpu.VMEM),
    )(eW, a, x, packed)


# speedup vs baseline: 1.6705x; 1.5904x over previous
import jax
import jax.numpy as jnp
from jax.experimental import pallas as pl
from jax.experimental.pallas import tpu as pltpu

B, N, F_IN, S, CH = 4, 64, 32, 4, 32
E = B * N * N


def _net_kernel(eW_ref, a_ref, x_ref,
                c1_w0_ref, c1_b0_ref, c1_w1_ref, c1_b1_ref, c1_w2_ref,
                c1_b2_ref, c1_root_ref, c1_bias_ref,
                c2_w0_ref, c2_b0_ref, c2_w1_ref, c2_b1_ref, c2_w2_ref,
                c2_b2_ref, c2_root_ref, c2_bias_ref,
                d_w_ref, d_b_ref, o_w_ref, o_b_ref,
                out_ref):
    eW = eW_ref[:]
    a = a_ref[:]
    xf = x_ref[:, :, :F_IN].reshape(B * N, F_IN)
    mask = x_ref[:, :, F_IN:F_IN + 1]
    bf = jnp.bfloat16

    aT = jnp.transpose(a, (0, 2, 1)).reshape(B * N, N)

    ri = jax.lax.broadcasted_iota(jnp.int32, (32 * S, 32, 64), 0) // S
    ci = jax.lax.broadcasted_iota(jnp.int32, (32 * S, 32, 64), 1)

    def block_diag_w0(w0):
        w0big = jnp.broadcast_to(jnp.tile(w0, (32, 1))[:, None, :],
                                 (32 * S, 32, 64))
        return jnp.where(ri == ci, w0big, 0.0)

    def ecc(feats, w0, b0, w1, b1, w2jcf, b2cf, root, bias):
        h1 = jax.nn.relu(
            jax.lax.dot_general(
                eW.astype(bf), block_diag_w0(w0).astype(bf),
                (((1,), (0,)), ((), ())),
                preferred_element_type=jnp.float32).reshape(E, 64)
            + b0)
        h2 = jax.nn.relu(
            jax.lax.dot(h1.astype(bf), w1.astype(bf),
                        preferred_element_type=jnp.float32)
            + b1)
        H = jnp.swapaxes(h2.astype(bf).reshape(B, N, N, 32), 1, 2)
        H = H.reshape(B * N, N, 32)
        G = jax.lax.dot_general(
            feats.astype(bf), w2jcf.astype(bf), (((1,), (2,)), ((), ())),
            preferred_element_type=jnp.float32)
        U = jax.lax.dot_general(
            G.astype(bf), H, (((1,), (2,)), ((0,), (0,))),
            preferred_element_type=jnp.float32)
        Um = U * aT[:, None, :]
        msg = Um.reshape(B, N, CH, N).sum(axis=1)
        msg = jnp.transpose(msg, (0, 2, 1))
        bterm = jax.lax.dot_general(
            feats, b2cf, (((1,), (1,)), ((), ())),
            preferred_element_type=jnp.float32)
        bmat = jax.lax.dot_general(
            a, bterm.reshape(B, N, CH), (((2,), (1,)), ((0,), (0,))),
            preferred_element_type=jnp.float32)
        rootterm = jax.lax.dot(
            feats, root, preferred_element_type=jnp.float32)
        return msg + bmat + rootterm.reshape(B, N, CH) + bias

    h = ecc(xf, c1_w0_ref[:], c1_b0_ref[:], c1_w1_ref[:], c1_b1_ref[:],
            c1_w2_ref[:], c1_b2_ref[:], c1_root_ref[:], c1_bias_ref[:])
    h = jnp.where(h > 0, h, 0.05 * h)
    h = ecc(h.reshape(B * N, CH),
            c2_w0_ref[:], c2_b0_ref[:], c2_w1_ref[:], c2_b1_ref[:],
            c2_w2_ref[:], c2_b2_ref[:], c2_root_ref[:], c2_bias_ref[:])
    h = jax.nn.relu(h)

    denom = jnp.clip(jnp.sum(mask, axis=1), 1.0, None)
    pooled = jnp.sum(h * mask, axis=1) / denom
    dh = jax.nn.relu(
        jax.lax.dot(pooled, d_w_ref[:], preferred_element_type=jnp.float32)
        + d_b_ref[:])
    logits = (jax.lax.dot(dh, o_w_ref[:], preferred_element_type=jnp.float32)
              + o_b_ref[:])
    m = jnp.max(logits, axis=-1, keepdims=True)
    ex = jnp.exp(logits - m)
    out_ref[:] = ex / jnp.sum(ex, axis=-1, keepdims=True)


def kernel(x, a, e, c1_w0, c1_b0, c1_w1, c1_b1, c1_w2, c1_b2, c1_root,
           c1_bias, c2_w0, c2_b0, c2_w1, c2_b1, c2_w2, c2_b2, c2_root,
           c2_bias, d_w, d_b, o_w, o_b):
    args = (
        e.reshape(E // 32, 32 * S), a, x,
        c1_w0, c1_b0.reshape(1, 64), c1_w1, c1_b1.reshape(1, 32),
        c1_w2.reshape(32, CH, F_IN), c1_b2.reshape(CH, F_IN),
        c1_root, c1_bias.reshape(1, CH),
        c2_w0, c2_b0.reshape(1, 64), c2_w1, c2_b1.reshape(1, 32),
        c2_w2.reshape(32, CH, CH), c2_b2.reshape(CH, CH),
        c2_root, c2_bias.reshape(1, CH),
        d_w, d_b.reshape(1, 64), o_w, o_b.reshape(1, 10),
    )
    return pl.pallas_call(
        _net_kernel,
        out_shape=jax.ShapeDtypeStruct((B, 10), jnp.float32),
        in_specs=[pl.BlockSpec(memory_space=pltpu.VMEM)] * len(args),
        out_specs=pl.BlockSpec(memory_space=pltpu.VMEM),
    )(*args)
